# Initial kernel scaffold; baseline (speedup 1.0000x reference)
import jax
import jax.numpy as jnp
from jax.experimental import pallas as pl


def kernel(bitvecs, tables):
    raise NotImplementedError("write your pallas kernel here")


# speedup vs baseline: 14.4227x; 14.4227x over previous
import functools

import jax
import jax.numpy as jnp
from jax import lax
from jax.experimental import pallas as pl
from jax.experimental.pallas import tpu as pltpu
from jax.experimental.pallas import tpu_sc as plsc

NUM_BITS = 8
EMB_DIM = 128
NUM_CODES = 1 << NUM_BITS


def _lut_body(tables_ref, bits_ref, lut_ref):
    t0 = tables_ref[0]
    t1 = tables_ref[1]
    diff = t1 - t0
    base = jnp.sum(t0, axis=0, keepdims=True)
    lut_ref[...] = (
        jnp.dot(bits_ref[...], diff, preferred_element_type=jnp.float32,
                precision=lax.Precision.HIGHEST) + base
    )


def _build_lut(tables_t, bits256):
    return pl.pallas_call(
        _lut_body,
        out_shape=jax.ShapeDtypeStruct((NUM_CODES, EMB_DIM), jnp.float32),
    )(tables_t, bits256)


def _make_sc_kernel(batch):
    info = plsc.get_sparse_core_info()
    nc, ns, lanes = info.num_cores, info.num_subcores, info.num_lanes
    nw = nc * ns
    b_per_w = batch // nw
    n_groups = b_per_w // lanes
    idx_chunk = 128
    n_chunks = b_per_w // idx_chunk

    mesh = plsc.VectorSubcoreMesh(core_axis_name="c", subcore_axis_name="s")

    @functools.partial(
        pl.kernel,
        mesh=mesh,
        out_type=jax.ShapeDtypeStruct((batch, EMB_DIM), jnp.float32),
        scratch_types=[
            pltpu.VMEM((NUM_BITS, b_per_w), jnp.int32),
            pltpu.VMEM((n_chunks, idx_chunk), jnp.int32),
            pltpu.VMEM((b_per_w, EMB_DIM), jnp.float32),
            pltpu.SemaphoreType.DMA,
        ],
    )
    def sc_kernel(bits_hbm, lut_hbm, out_hbm, bits_v, code_v, rows_v, sem):
        wid = lax.axis_index("s") * nc + lax.axis_index("c")
        base = wid * b_per_w
        pltpu.sync_copy(bits_hbm.at[wid], bits_v)
        for g in range(n_groups):
            code = bits_v[0, pl.ds(g * lanes, lanes)]
            for i in range(1, NUM_BITS):
                code = code + (bits_v[i, pl.ds(g * lanes, lanes)] << i)
            code_v[g // (idx_chunk // lanes),
                   pl.ds((g % (idx_chunk // lanes)) * lanes, lanes)] = code
        copies = [
            pltpu.async_copy(
                lut_hbm.at[code_v.at[j]],
                rows_v.at[pl.ds(j * idx_chunk, idx_chunk)],
                sem,
            )
            for j in range(n_chunks)
        ]
        for c in copies:
            c.wait()
        pltpu.sync_copy(rows_v, out_hbm.at[pl.ds(base, b_per_w)])

    return sc_kernel


@jax.jit
def kernel(bitvecs, tables):
    batch = bitvecs.shape[0]
    info = plsc.get_sparse_core_info()
    nw = info.num_cores * info.num_subcores
    b_per_w = batch // nw

    bits_i32 = bitvecs.astype(jnp.int32)
    bits_blocks = bits_i32.T.reshape(NUM_BITS, nw, b_per_w).transpose(1, 0, 2)
    tables_t = tables.astype(jnp.float32).transpose(1, 0, 2)

    codes = lax.iota(jnp.int32, NUM_CODES)[:, None]
    shifts = lax.iota(jnp.int32, NUM_BITS)[None, :]
    bits256 = ((codes >> shifts) & 1).astype(jnp.float32)

    lut = _build_lut(tables_t, bits256)
    out = _make_sc_kernel(batch)(bits_blocks, lut)
    return out
